# baseline (speedup 1.0000x reference)
import functools
import jax
import jax.numpy as jnp
from jax.experimental import pallas as pl

_BLK = 400


def _proj_relu_kernel(x_ref, w_ref, b_ref, o_ref):
    acc = jnp.dot(x_ref[...], w_ref[...], preferred_element_type=jnp.float32)
    o_ref[...] = jnp.maximum(acc + b_ref[...], 0.0)


def _proj_relu(x, w, b):
    n, d_in = x.shape
    d_out = w.shape[1]
    grid = (n // _BLK,)
    return pl.pallas_call(
        _proj_relu_kernel,
        grid=grid,
        in_specs=[
            pl.BlockSpec((_BLK, d_in), lambda i: (i, 0)),
            pl.BlockSpec((d_in, d_out), lambda i: (0, 0)),
            pl.BlockSpec((1, d_out), lambda i: (0, 0)),
        ],
        out_specs=pl.BlockSpec((_BLK, d_out), lambda i: (i, 0)),
        out_shape=jax.ShapeDtypeStruct((n, d_out), jnp.float32),
    )(x, w, b.reshape(1, d_out))


def _rel_kernel(h_ref, w_ref, a_ref, hs_ref, s_ref):
    hs = jnp.dot(h_ref[...], w_ref[...], preferred_element_type=jnp.float32)
    hs_ref[...] = hs
    s_ref[...] = jnp.dot(hs, a_ref[...], preferred_element_type=jnp.float32)


def _rel_transform(h, w, a):
    n, d = h.shape
    grid = (n // _BLK,)
    return pl.pallas_call(
        _rel_kernel,
        grid=grid,
        in_specs=[
            pl.BlockSpec((_BLK, d), lambda i: (i, 0)),
            pl.BlockSpec((d, d), lambda i: (0, 0)),
            pl.BlockSpec((d, 1), lambda i: (0, 0)),
        ],
        out_specs=[
            pl.BlockSpec((_BLK, d), lambda i: (i, 0)),
            pl.BlockSpec((_BLK, 1), lambda i: (i, 0)),
        ],
        out_shape=[
            jax.ShapeDtypeStruct((n, d), jnp.float32),
            jax.ShapeDtypeStruct((n, 1), jnp.float32),
        ],
    )(h, w, a.reshape(d, 1))


def _head_kernel(x_ref, w_ref, b_ref, o_ref):
    acc = jnp.dot(x_ref[...], w_ref[...], preferred_element_type=jnp.float32)
    o_ref[...] = jax.nn.sigmoid(acc + b_ref[...])


def _head(x, w, b):
    n, d = x.shape
    d_out = w.shape[1]
    grid = (n // _BLK,)
    return pl.pallas_call(
        _head_kernel,
        grid=grid,
        in_specs=[
            pl.BlockSpec((_BLK, d), lambda i: (i, 0)),
            pl.BlockSpec((d, d_out), lambda i: (0, 0)),
            pl.BlockSpec((1, d_out), lambda i: (0, 0)),
        ],
        out_specs=pl.BlockSpec((_BLK, d_out), lambda i: (i, 0)),
        out_shape=jax.ShapeDtypeStruct((n, d_out), jnp.float32),
    )(x, w, b.reshape(1, d_out))


def kernel(x_bus, x_generator, edge_index_bus_bus, edge_index_gen_bus,
           edge_index_bus_gen, lin_bus_W, lin_bus_b, lin_gen_W, lin_gen_b,
           conv_W, conv_a_src, conv_a_dst, out_bus_W, out_bus_b,
           out_gen_W, out_gen_b):
    h = {
        'bus': _proj_relu(x_bus, lin_bus_W, lin_bus_b),
        'generator': _proj_relu(x_generator, lin_gen_W, lin_gen_b),
    }
    sizes = {'bus': x_bus.shape[0], 'generator': x_generator.shape[0]}
    rels = [('bus', 'bus', edge_index_bus_bus),
            ('generator', 'bus', edge_index_gen_bus),
            ('bus', 'generator', edge_index_bus_gen)]
    num_layers = conv_W.shape[0]
    d_h = conv_W.shape[-1]
    for l in range(num_layers):
        out = {k: jnp.zeros((sizes[k], d_h), jnp.float32) for k in h}
        for r, (st, dt, ei) in enumerate(rels):
            hs, ss = _rel_transform(h[st], conv_W[l, r], conv_a_src[l, r])
            if st == dt:
                sd = jnp.dot(hs, conv_a_dst[l, r])
            else:
                _, sd1 = _rel_transform(h[dt], conv_W[l, r], conv_a_dst[l, r])
                sd = sd1[:, 0]
            ss = ss[:, 0]
            src = ei[0]
            dst = ei[1]
            logits = ss[src] + sd[dst]
            logits = jnp.where(logits > 0, logits, 0.2 * logits)
            Nd = sizes[dt]
            m = jax.ops.segment_max(logits, dst, num_segments=Nd)
            m = jnp.where(jnp.isfinite(m), m, 0.0)
            ex = jnp.exp(logits - m[dst])
            den = jax.ops.segment_sum(ex, dst, num_segments=Nd)
            alpha = ex / (den[dst] + 1e-16)
            msg = hs[src] * alpha[:, None]
            out[dt] = out[dt] + jax.ops.segment_sum(msg, dst, num_segments=Nd)
        h = {k: jax.nn.relu(v) for k, v in out.items()}
    bus_out = _head(h['bus'], out_bus_W, out_bus_b)
    gen_out = _head(h['generator'], out_gen_W, out_gen_b)
    return (bus_out, gen_out)
